# baseline (speedup 1.0000x reference)
import functools

import jax
import jax.numpy as jnp
from jax import lax
from jax.experimental import pallas as pl
from jax.experimental.pallas import tpu as pltpu
from jax.experimental.pallas import tpu_sc as plsc

BATCH = 16384
D = 64
V = 1000000
NC = 2
NS = 16
NW = NC * NS
B_PER_W = BATCH // NW
CHUNK = 128
NCHUNK = B_PER_W // CHUNK
RB = 1536


def _transpose_body(t_ref, out_ref):
    r = lax.broadcasted_iota(jnp.int32, (D, D), 0)
    c = lax.broadcasted_iota(jnp.int32, (D, D), 1)
    eye = (r == c).astype(jnp.float32)
    out_ref[:, :D] = lax.dot_general(
        t_ref[...], eye, (((0,), (0,)), ((), ())),
        preferred_element_type=jnp.float32)


def _tc_pad_transpose(tab_t):
    return pl.pallas_call(
        _transpose_body,
        grid=(pl.cdiv(V, RB),),
        in_specs=[pl.BlockSpec((D, RB), lambda i: (0, i))],
        out_specs=pl.BlockSpec((RB, 128), lambda i: (i, 0)),
        out_shape=jax.ShapeDtypeStruct((V, 128), jnp.float32),
    )(tab_t)


def _gather_body(u_hbm, gp_hbm, gn_hbm, upad_hbm, ipairs_hbm,
                 wu_out, wp_out, wn_out,
                 gidx, rows, sem):
    cid = lax.axis_index("c")
    sid = lax.axis_index("s")
    wid = sid * NC + cid
    base = pl.multiple_of(wid * B_PER_W, B_PER_W)

    for idx_hbm, table_hbm, out in (
        (u_hbm, upad_hbm, wu_out),
        (gp_hbm, ipairs_hbm, wp_out),
        (gn_hbm, ipairs_hbm, wn_out),
    ):
        pltpu.sync_copy(idx_hbm.at[wid], gidx)
        copies = []
        for c in range(NCHUNK):
            copies.append(pltpu.async_copy(
                table_hbm.at[gidx.at[c]],
                rows.at[pl.ds(c * CHUNK, CHUNK), :], sem))
        for cp in copies:
            cp.wait()
        pltpu.sync_copy(rows, out.at[pl.ds(base, B_PER_W), :])


@jax.jit
def _bprmf_call(user, pos_item, neg_item, user_table, item_table):
    upad = _tc_pad_transpose(user_table.T)
    ipairs = item_table.reshape(V // 2, 2 * D)

    mesh = plsc.VectorSubcoreMesh(core_axis_name="c", subcore_axis_name="s")
    out_w = jax.ShapeDtypeStruct((BATCH, 2 * D), jnp.float32)
    fn = functools.partial(
        pl.kernel,
        mesh=mesh,
        out_type=(out_w, out_w, out_w),
        scratch_types=[
            pltpu.VMEM((NCHUNK, CHUNK), jnp.int32),
            pltpu.VMEM((B_PER_W, 2 * D), jnp.float32),
            pltpu.SemaphoreType.DMA,
        ],
    )(_gather_body)
    u_r = user.reshape(NW, NCHUNK, CHUNK)
    gp = (pos_item >> 1).reshape(NW, NCHUNK, CHUNK)
    gn = (neg_item >> 1).reshape(NW, NCHUNK, CHUNK)
    wu, wp, wn = fn(u_r, gp, gn, upad, ipairs)

    def half_select(wide, idx):
        odd = (idx & 1).astype(jnp.bool_)[:, None]
        return jnp.where(odd, wide[:, D:], wide[:, :D])

    return (wu[:, :D],
            half_select(wp, pos_item),
            half_select(wn, neg_item))


def kernel(user, pos_item, neg_item, user_table, item_table):
    return _bprmf_call(user, pos_item, neg_item, user_table, item_table)


# speedup vs baseline: 1.2126x; 1.2126x over previous
import functools

import jax
import jax.numpy as jnp
from jax import lax
from jax.experimental import pallas as pl
from jax.experimental.pallas import tpu as pltpu
from jax.experimental.pallas import tpu_sc as plsc

BATCH = 16384
D = 64
V = 1000000
NC = 2
NS = 16
NW = NC * NS
B_PER_W = BATCH // NW
CHUNK = 128
NCHUNK = B_PER_W // CHUNK
RB = 6144


def _transpose_body(t_ref, out_ref):
    r = lax.broadcasted_iota(jnp.int32, (D, D), 0)
    c = lax.broadcasted_iota(jnp.int32, (D, D), 1)
    eye = (r == c).astype(jnp.float32)
    tr = lax.dot_general(
        t_ref[...], eye, (((0,), (0,)), ((), ())),
        preferred_element_type=jnp.float32)
    out_ref[...] = jnp.concatenate((tr, tr), axis=1)


def _tc_pad_transpose(tab_t):
    return pl.pallas_call(
        _transpose_body,
        grid=(pl.cdiv(V, RB),),
        in_specs=[pl.BlockSpec((D, RB), lambda i: (0, i))],
        out_specs=pl.BlockSpec((RB, 128), lambda i: (i, 0)),
        out_shape=jax.ShapeDtypeStruct((V, 128), jnp.float32),
    )(tab_t)


def _gather_body(u_hbm, gp_hbm, gn_hbm, upad_hbm, ipairs_hbm,
                 wu_out, wp_out, wn_out,
                 gidx, rows, sem):
    cid = lax.axis_index("c")
    sid = lax.axis_index("s")
    wid = sid * NC + cid
    base = pl.multiple_of(wid * B_PER_W, B_PER_W)

    for idx_hbm, table_hbm, out in (
        (u_hbm, upad_hbm, wu_out),
        (gp_hbm, ipairs_hbm, wp_out),
        (gn_hbm, ipairs_hbm, wn_out),
    ):
        pltpu.sync_copy(idx_hbm.at[wid], gidx)
        copies = []
        for c in range(NCHUNK):
            copies.append(pltpu.async_copy(
                table_hbm.at[gidx.at[c]],
                rows.at[pl.ds(c * CHUNK, CHUNK), :], sem))
        for cp in copies:
            cp.wait()
        pltpu.sync_copy(rows, out.at[pl.ds(base, B_PER_W), :])


@jax.jit
def _bprmf_call(user, pos_item, neg_item, user_table, item_table):
    upad = _tc_pad_transpose(user_table.T)
    ipairs = item_table.reshape(V // 2, 2 * D)

    mesh = plsc.VectorSubcoreMesh(core_axis_name="c", subcore_axis_name="s")
    out_w = jax.ShapeDtypeStruct((BATCH, 2 * D), jnp.float32)
    fn = functools.partial(
        pl.kernel,
        mesh=mesh,
        out_type=(out_w, out_w, out_w),
        scratch_types=[
            pltpu.VMEM((NCHUNK, CHUNK), jnp.int32),
            pltpu.VMEM((B_PER_W, 2 * D), jnp.float32),
            pltpu.SemaphoreType.DMA,
        ],
    )(_gather_body)
    u_r = user.reshape(NW, NCHUNK, CHUNK)
    gp = (pos_item >> 1).reshape(NW, NCHUNK, CHUNK)
    gn = (neg_item >> 1).reshape(NW, NCHUNK, CHUNK)
    wu, wp, wn = fn(u_r, gp, gn, upad, ipairs)

    def half_select(wide, idx):
        odd = (idx & 1).astype(jnp.bool_)[:, None]
        return jnp.where(odd, wide[:, D:], wide[:, :D])

    return (wu[:, :D],
            half_select(wp, pos_item),
            half_select(wn, neg_item))


def kernel(user, pos_item, neg_item, user_table, item_table):
    return _bprmf_call(user, pos_item, neg_item, user_table, item_table)
